# Initial kernel scaffold; baseline (speedup 1.0000x reference)
import functools

import jax
import jax.numpy as jnp
from jax import lax
from jax.experimental import pallas as pl
from jax.experimental.pallas import tpu as pltpu
from jax.experimental.pallas import tpu_sc as plsc

N = 10000
E = 320000
D = 128
K = 5000
NPAD = 16384
KPAD = 5120
NC, NS = 2, 16
NW = NC * NS
ROWS_PW = KPAD // NW
EDGES_PW = E // NW


def _bitonic_stage(v, idx, i2, k, j):
    if j < 128:
        pv = jnp.where((i2 & j) == 0, jnp.roll(v, -j, axis=1), jnp.roll(v, j, axis=1))
        pi = jnp.where((i2 & j) == 0, jnp.roll(idx, -j, axis=1), jnp.roll(idx, j, axis=1))
    else:
        jr = j // 128
        pv = jnp.where((i2 & j) == 0, jnp.roll(v, -jr, axis=0), jnp.roll(v, jr, axis=0))
        pi = jnp.where((i2 & j) == 0, jnp.roll(idx, -jr, axis=0), jnp.roll(idx, jr, axis=0))
    low = (i2 & j) == 0
    asc = (i2 & k) == 0
    pm = (v > pv) | ((v == pv) & (idx < pi))
    take = jnp.logical_xor(pm, jnp.logical_xor(low, asc))
    return jnp.where(take, v, pv), jnp.where(take, idx, pi)


def _sort_body(s_ref, vals_ref, ids_ref):
    v = s_ref[...]
    r2 = lax.broadcasted_iota(jnp.int32, (128, 128), 0)
    c2 = lax.broadcasted_iota(jnp.int32, (128, 128), 1)
    i2 = r2 * 128 + c2
    idx = i2
    k = 2
    while k <= NPAD:
        j = k // 2
        while j >= 1:
            v, idx = _bitonic_stage(v, idx, i2, k, j)
            j //= 2
        k *= 2
    vals_ref[...] = v
    ids_ref[...] = idx


def _topk_sort(spad2d, interpret=False):
    return pl.pallas_call(
        _sort_body,
        out_shape=(
            jax.ShapeDtypeStruct((128, 128), jnp.float32),
            jax.ShapeDtypeStruct((128, 128), jnp.int32),
        ),
        interpret=interpret,
    )(spad2d)


def _sc_gpool_body(h_hbm, ids_hbm, vals_hbm, ei_hbm, ks_hbm,
                   newh_hbm, ss_hbm, sd_hbm, kp_hbm,
                   inv_v, ids_v, myids_v, vals_v, rows_v,
                   src_v, dst_v, ssv, sdv, kpv, ks_v, sem):
    cid = lax.axis_index("c")
    sid = lax.axis_index("s")
    wid = sid * NC + cid

    pltpu.sync_copy(ids_hbm, ids_v)
    pltpu.sync_copy(ks_hbm, ks_v)
    base = wid * ROWS_PW
    pltpu.sync_copy(vals_hbm.at[pl.ds(base, ROWS_PW)], vals_v)
    pltpu.sync_copy(ids_hbm.at[pl.ds(base, 80)], myids_v.at[0])
    pltpu.sync_copy(ids_hbm.at[pl.ds(base + 80, 80)], myids_v.at[1])
    ebase = wid * EDGES_PW
    pltpu.sync_copy(ei_hbm.at[0, pl.ds(ebase, EDGES_PW)], src_v)
    pltpu.sync_copy(ei_hbm.at[1, pl.ds(ebase, EDGES_PW)], dst_v)

    g0 = pltpu.async_copy(h_hbm.at[myids_v.at[0]], rows_v.at[pl.ds(0, 80)], sem)
    g1 = pltpu.async_copy(h_hbm.at[myids_v.at[1]], rows_v.at[pl.ds(80, 80)], sem)

    neg1 = jnp.full((16,), -1, jnp.int32)

    def init_body(t, carry):
        inv_v[pl.ds(t * 16, 16)] = neg1
        return carry

    lax.fori_loop(0, N // 16, init_body, 0)

    iota16 = lax.iota(jnp.int32, 16)

    def scat_body(t, carry):
        ids16 = ids_v[pl.ds(t * 16, 16)]
        ranks = iota16 + t * 16
        plsc.store_scatter(inv_v, [ids16], ranks, mask=ranks < K)
        return carry

    lax.fori_loop(0, (K + 15) // 16, scat_body, 0)

    g0.wait()
    g1.wait()

    def scale_body(j, carry):
        jv = jnp.full((16,), 0, jnp.int32) + j
        bv = plsc.load_gather(vals_v, [jv])
        for c in range(D // 16):
            rows_v[j, pl.ds(c * 16, 16)] = rows_v[j, pl.ds(c * 16, 16)] * bv
        return carry

    lax.fori_loop(0, ROWS_PW, scale_body, 0)
    pltpu.sync_copy(rows_v, newh_hbm.at[pl.ds(base, ROWS_PW)])

    kvec = ks_v[...]

    def edge_body(t, carry):
        s16 = src_v[pl.ds(t * 16, 16)]
        d16 = dst_v[pl.ds(t * 16, 16)]
        invs = plsc.load_gather(inv_v, [s16])
        invd = plsc.load_gather(inv_v, [d16])
        keep = (invs >= 0) & (invd >= 0)
        ssv[pl.ds(t * 16, 16)] = jnp.where(keep, invs + kvec, -1)
        sdv[pl.ds(t * 16, 16)] = jnp.where(keep, invd + kvec, -1)
        kpv[pl.ds(t * 16, 16)] = keep.astype(jnp.int32)
        return carry

    lax.fori_loop(0, EDGES_PW // 16, edge_body, 0)
    pltpu.sync_copy(ssv, ss_hbm.at[pl.ds(ebase, EDGES_PW)])
    pltpu.sync_copy(sdv, sd_hbm.at[pl.ds(ebase, EDGES_PW)])
    pltpu.sync_copy(kpv, kp_hbm.at[pl.ds(ebase, EDGES_PW)])


def _sc_gpool(h, ids_pad, vals_pad, edge_index, ks16, interpret=False):
    mesh = plsc.VectorSubcoreMesh(
        core_axis_name="c", subcore_axis_name="s", num_cores=NC, num_subcores=NS)
    fn = functools.partial(
        pl.kernel,
        out_type=[
            jax.ShapeDtypeStruct((KPAD, D), jnp.float32),
            jax.ShapeDtypeStruct((E,), jnp.int32),
            jax.ShapeDtypeStruct((E,), jnp.int32),
            jax.ShapeDtypeStruct((E,), jnp.int32),
        ],
        mesh=mesh,
        scratch_types=[
            pltpu.VMEM((N,), jnp.int32),
            pltpu.VMEM((KPAD,), jnp.int32),
            pltpu.VMEM((2, 80), jnp.int32),
            pltpu.VMEM((ROWS_PW,), jnp.float32),
            pltpu.VMEM((ROWS_PW, D), jnp.float32),
            pltpu.VMEM((EDGES_PW,), jnp.int32),
            pltpu.VMEM((EDGES_PW,), jnp.int32),
            pltpu.VMEM((EDGES_PW,), jnp.int32),
            pltpu.VMEM((EDGES_PW,), jnp.int32),
            pltpu.VMEM((EDGES_PW,), jnp.int32),
            pltpu.VMEM((16,), jnp.int32),
            pltpu.SemaphoreType.DMA,
        ],
        interpret=interpret,
    )(_sc_gpool_body)
    return fn(h, ids_pad, vals_pad, edge_index, ks16)


def kernel(h, edge_index, top_k, W, b):
    scores = jax.nn.sigmoid(h @ W + b)[:, 0]
    k_shift = jnp.asarray(top_k, jnp.int32) - jnp.int32(K)

    spad = jnp.concatenate(
        [scores, jnp.full((NPAD - N,), -1.0, jnp.float32)]).reshape(128, 128)
    vals2d, ids2d = _topk_sort(spad)
    vals_lin = vals2d.reshape(-1)
    ids_lin = ids2d.reshape(-1)
    node_ids = ids_lin[:K]

    ks16 = jnp.full((16,), 1, jnp.int32) * k_shift
    newh_pad, ss, sd, kp = _sc_gpool(
        h, ids_lin[:KPAD], vals_lin[:KPAD], edge_index, ks16)

    return (newh_pad[:K], node_ids, ss, sd, kp.astype(jnp.bool_))


# speedup vs baseline: 138.0571x; 138.0571x over previous
import functools

import jax
import jax.numpy as jnp
from jax import lax
from jax.experimental import pallas as pl
from jax.experimental.pallas import tpu as pltpu
from jax.experimental.pallas import tpu_sc as plsc

N = 10000
E = 320000
D = 128
K = 5000
NPAD = 16384
KPAD = 5120
NC, NS = 2, 16
NW = NC * NS
ROWS_PW = KPAD // NW
EDGES_PW = E // NW


def _bitonic_stage(v, idx, i2, k, j):
    if j < 128:
        pv = jnp.where((i2 & j) == 0, jnp.roll(v, -j, axis=1), jnp.roll(v, j, axis=1))
        pi = jnp.where((i2 & j) == 0, jnp.roll(idx, -j, axis=1), jnp.roll(idx, j, axis=1))
    else:
        jr = j // 128
        pv = jnp.where((i2 & j) == 0, jnp.roll(v, -jr, axis=0), jnp.roll(v, jr, axis=0))
        pi = jnp.where((i2 & j) == 0, jnp.roll(idx, -jr, axis=0), jnp.roll(idx, jr, axis=0))
    low = (i2 & j) == 0
    asc = (i2 & k) == 0
    pm = (v > pv) | ((v == pv) & (idx < pi))
    take = jnp.logical_xor(pm, jnp.logical_xor(low, asc))
    return jnp.where(take, v, pv), jnp.where(take, idx, pi)


def _sort_body(s_ref, vals_ref, ids_ref):
    v = s_ref[...]
    r2 = lax.broadcasted_iota(jnp.int32, (128, 128), 0)
    c2 = lax.broadcasted_iota(jnp.int32, (128, 128), 1)
    i2 = r2 * 128 + c2
    idx = i2
    k = 2
    while k <= NPAD:
        j = k // 2
        while j >= 1:
            v, idx = _bitonic_stage(v, idx, i2, k, j)
            j //= 2
        k *= 2
    vals_ref[...] = v
    ids_ref[...] = idx


def _topk_sort(spad2d, interpret=False):
    return pl.pallas_call(
        _sort_body,
        out_shape=(
            jax.ShapeDtypeStruct((128, 128), jnp.float32),
            jax.ShapeDtypeStruct((128, 128), jnp.int32),
        ),
        interpret=interpret,
    )(spad2d)


def _sc_gpool_body(h_hbm, ids_hbm, vals_hbm, ei_hbm, ks_hbm,
                   newh_hbm, ss_hbm, sd_hbm, kp_hbm,
                   inv_v, ids_v, myids_v, vals_v, rows_v,
                   src_v, dst_v, ssv, sdv, kpv, ks_v, sem):
    cid = lax.axis_index("c")
    sid = lax.axis_index("s")
    wid = sid * NC + cid

    pltpu.sync_copy(ids_hbm, ids_v)
    pltpu.sync_copy(ks_hbm, ks_v)
    base = wid * ROWS_PW
    pltpu.sync_copy(vals_hbm.at[pl.ds(base, ROWS_PW)], vals_v)
    pltpu.sync_copy(ids_hbm.at[pl.ds(base, 80)], myids_v.at[0])
    pltpu.sync_copy(ids_hbm.at[pl.ds(base + 80, 80)], myids_v.at[1])
    ebase = wid * EDGES_PW
    pltpu.sync_copy(ei_hbm.at[pl.ds(ebase, EDGES_PW)], src_v)
    pltpu.sync_copy(ei_hbm.at[pl.ds(E + ebase, EDGES_PW)], dst_v)

    g0 = pltpu.async_copy(h_hbm.at[myids_v.at[0]], rows_v.at[pl.ds(0, 80)], sem)
    g1 = pltpu.async_copy(h_hbm.at[myids_v.at[1]], rows_v.at[pl.ds(80, 80)], sem)

    neg1 = jnp.full((16,), -1, jnp.int32)

    def init_body(t, carry):
        inv_v[pl.ds(t * 16, 16)] = neg1
        return carry

    lax.fori_loop(0, N // 16, init_body, 0)

    iota16 = lax.iota(jnp.int32, 16)

    def scat_body(t, carry):
        ids16 = ids_v[pl.ds(t * 16, 16)]
        ranks = iota16 + t * 16
        plsc.store_scatter(inv_v, [ids16], ranks, mask=ranks < K)
        return carry

    lax.fori_loop(0, (K + 15) // 16, scat_body, 0)

    g0.wait()
    g1.wait()

    def scale_body(j, carry):
        jv = jnp.full((16,), 0, jnp.int32) + j
        bv = plsc.load_gather(vals_v, [jv])
        for c in range(D // 16):
            rows_v[j, pl.ds(c * 16, 16)] = rows_v[j, pl.ds(c * 16, 16)] * bv
        return carry

    lax.fori_loop(0, ROWS_PW, scale_body, 0)
    pltpu.sync_copy(rows_v, newh_hbm.at[pl.ds(base, ROWS_PW)])

    kvec = ks_v[...]

    def edge_body(t, carry):
        s16 = src_v[pl.ds(t * 16, 16)]
        d16 = dst_v[pl.ds(t * 16, 16)]
        invs = plsc.load_gather(inv_v, [s16])
        invd = plsc.load_gather(inv_v, [d16])
        keep = (invs >= 0) & (invd >= 0)
        ssv[pl.ds(t * 16, 16)] = jnp.where(keep, invs + kvec, -1)
        sdv[pl.ds(t * 16, 16)] = jnp.where(keep, invd + kvec, -1)
        kpv[pl.ds(t * 16, 16)] = keep.astype(jnp.int32)
        return carry

    lax.fori_loop(0, EDGES_PW // 16, edge_body, 0)
    pltpu.sync_copy(ssv, ss_hbm.at[pl.ds(ebase, EDGES_PW)])
    pltpu.sync_copy(sdv, sd_hbm.at[pl.ds(ebase, EDGES_PW)])
    pltpu.sync_copy(kpv, kp_hbm.at[pl.ds(ebase, EDGES_PW)])


def _sc_gpool(h, ids_pad, vals_pad, ei_flat, ks16, interpret=False):
    mesh = plsc.VectorSubcoreMesh(
        core_axis_name="c", subcore_axis_name="s", num_cores=NC, num_subcores=NS)
    fn = functools.partial(
        pl.kernel,
        out_type=[
            jax.ShapeDtypeStruct((KPAD, D), jnp.float32),
            jax.ShapeDtypeStruct((E,), jnp.int32),
            jax.ShapeDtypeStruct((E,), jnp.int32),
            jax.ShapeDtypeStruct((E,), jnp.int32),
        ],
        mesh=mesh,
        compiler_params=pltpu.CompilerParams(needs_layout_passes=False),
        scratch_types=[
            pltpu.VMEM((N,), jnp.int32),
            pltpu.VMEM((KPAD,), jnp.int32),
            pltpu.VMEM((2, 80), jnp.int32),
            pltpu.VMEM((ROWS_PW,), jnp.float32),
            pltpu.VMEM((ROWS_PW, D), jnp.float32),
            pltpu.VMEM((EDGES_PW,), jnp.int32),
            pltpu.VMEM((EDGES_PW,), jnp.int32),
            pltpu.VMEM((EDGES_PW,), jnp.int32),
            pltpu.VMEM((EDGES_PW,), jnp.int32),
            pltpu.VMEM((EDGES_PW,), jnp.int32),
            pltpu.VMEM((16,), jnp.int32),
            pltpu.SemaphoreType.DMA,
        ],
        interpret=interpret,
    )(_sc_gpool_body)
    return fn(h, ids_pad, vals_pad, ei_flat, ks16)


def kernel(h, edge_index, top_k, W, b):
    scores = jax.nn.sigmoid(h @ W + b)[:, 0]
    k_shift = jnp.asarray(top_k, jnp.int32) - jnp.int32(K)

    spad = jnp.concatenate(
        [scores, jnp.full((NPAD - N,), -1.0, jnp.float32)]).reshape(128, 128)
    vals2d, ids2d = _topk_sort(spad)
    vals_lin = vals2d.reshape(-1)
    ids_lin = ids2d.reshape(-1)
    node_ids = ids_lin[:K]

    ks16 = jnp.full((16,), 1, jnp.int32) * k_shift
    newh_pad, ss, sd, kp = _sc_gpool(
        h, ids_lin[:KPAD], vals_lin[:KPAD], edge_index.reshape(-1), ks16)

    return (newh_pad[:K], node_ids, ss, sd, kp.astype(jnp.bool_))


# speedup vs baseline: 174.7869x; 1.2660x over previous
import functools

import jax
import jax.numpy as jnp
from jax import lax
from jax.experimental import pallas as pl
from jax.experimental.pallas import tpu as pltpu
from jax.experimental.pallas import tpu_sc as plsc

N = 10000
E = 320000
D = 128
K = 5000
NPAD = 16384
KPAD = 5120
NC, NS = 2, 16
NW = NC * NS
ROWS_PW = KPAD // NW
EDGES_PW = E // NW


def _axis_mask(bit, axis):
    ax = lax.broadcasted_iota(jnp.int32, (128, 128), axis)
    return (ax & bit) == 0


def _bitonic_stage(v, idx, k, j):
    low = _axis_mask(j, 1) if j < 128 else _axis_mask(j // 128, 0)
    if j < 128:
        pv = jnp.where(low, jnp.roll(v, -j, axis=1), jnp.roll(v, j, axis=1))
        pi = jnp.where(low, jnp.roll(idx, -j, axis=1), jnp.roll(idx, j, axis=1))
    else:
        jr = j // 128
        pv = jnp.where(low, jnp.roll(v, -jr, axis=0), jnp.roll(v, jr, axis=0))
        pi = jnp.where(low, jnp.roll(idx, -jr, axis=0), jnp.roll(idx, jr, axis=0))
    asc = _axis_mask(k, 1) if k < 128 else _axis_mask(k // 128, 0)
    pm = (v > pv) | ((v == pv) & (idx < pi))
    take = jnp.logical_xor(pm, jnp.logical_xor(low, asc))
    return jnp.where(take, v, pv), jnp.where(take, idx, pi)


def _sort_body(s_ref, vals_ref, ids_ref):
    v = s_ref[...]
    r2 = lax.broadcasted_iota(jnp.int32, (128, 128), 0)
    c2 = lax.broadcasted_iota(jnp.int32, (128, 128), 1)
    idx = r2 * 128 + c2
    k = 2
    while k <= NPAD:
        j = k // 2
        while j >= 1:
            v, idx = _bitonic_stage(v, idx, k, j)
            j //= 2
        k *= 2
    vals_ref[...] = v
    ids_ref[...] = idx


def _topk_sort(spad2d, interpret=False):
    return pl.pallas_call(
        _sort_body,
        out_shape=(
            jax.ShapeDtypeStruct((128, 128), jnp.float32),
            jax.ShapeDtypeStruct((128, 128), jnp.int32),
        ),
        interpret=interpret,
    )(spad2d)


def _sc_gpool_body(h_hbm, ids_hbm, vals_hbm, ei_hbm, ks_hbm,
                   newh_hbm, ss_hbm, sd_hbm, kp_hbm,
                   inv_v, ids_v, myids_v, vals_v, rows_v,
                   src_v, dst_v, ssv, sdv, kpv, ks_v,
                   sem_m, sem_i, sem_v, sem_e, sem_g, sem_o):
    cid = lax.axis_index("c")
    sid = lax.axis_index("s")
    wid = sid * NC + cid
    base = wid * ROWS_PW
    ebase = wid * EDGES_PW

    a_m0 = pltpu.async_copy(ids_hbm.at[pl.ds(base, 80)], myids_v.at[0], sem_m)
    a_m1 = pltpu.async_copy(ids_hbm.at[pl.ds(base + 80, 80)], myids_v.at[1], sem_m)
    a_ids = pltpu.async_copy(ids_hbm, ids_v, sem_i)
    a_vals = pltpu.async_copy(vals_hbm.at[pl.ds(base, ROWS_PW)], vals_v, sem_v)
    a_ks = pltpu.async_copy(ks_hbm, ks_v, sem_v)
    a_src = pltpu.async_copy(ei_hbm.at[pl.ds(ebase, EDGES_PW)], src_v, sem_e)
    a_dst = pltpu.async_copy(ei_hbm.at[pl.ds(E + ebase, EDGES_PW)], dst_v, sem_e)

    neg1 = jnp.full((16,), -1, jnp.int32)

    @plsc.parallel_loop(0, N // 16, unroll=5)
    def _init(t):
        inv_v[pl.ds(t * 16, 16)] = neg1

    a_m0.wait()
    a_m1.wait()
    g0 = pltpu.async_copy(h_hbm.at[myids_v.at[0]], rows_v.at[pl.ds(0, 80)], sem_g)
    g1 = pltpu.async_copy(h_hbm.at[myids_v.at[1]], rows_v.at[pl.ds(80, 80)], sem_g)

    a_ids.wait()
    iota16 = lax.iota(jnp.int32, 16)

    @plsc.parallel_loop(0, 316, unroll=4)
    def _scat(t):
        ids16 = ids_v[pl.ds(t * 16, 16)]
        ranks = iota16 + t * 16
        plsc.store_scatter(inv_v, [ids16], ranks, mask=ranks < K)

    g0.wait()
    g1.wait()
    a_vals.wait()
    a_ks.wait()

    @plsc.parallel_loop(0, ROWS_PW, unroll=2)
    def _scale(j):
        jv = jnp.full((16,), 0, jnp.int32) + j
        bv = plsc.load_gather(vals_v, [jv])
        for c in range(D // 16):
            rows_v[j, pl.ds(c * 16, 16)] = rows_v[j, pl.ds(c * 16, 16)] * bv

    @pl.when(wid < NW - 1)
    def _():
        pltpu.sync_copy(rows_v, newh_hbm.at[pl.ds(base, ROWS_PW)])

    @pl.when(wid == NW - 1)
    def _():
        pltpu.sync_copy(rows_v.at[pl.ds(0, K - (NW - 1) * ROWS_PW)],
                        newh_hbm.at[pl.ds(base, K - (NW - 1) * ROWS_PW)])

    kvec = ks_v[...]
    a_src.wait()
    a_dst.wait()

    @plsc.parallel_loop(0, EDGES_PW // 16, unroll=5)
    def _edge(t):
        s16 = src_v[pl.ds(t * 16, 16)]
        d16 = dst_v[pl.ds(t * 16, 16)]
        invs = plsc.load_gather(inv_v, [s16])
        invd = plsc.load_gather(inv_v, [d16])
        m = (invs | invd) >> 31
        ssv[pl.ds(t * 16, 16)] = (invs + kvec) | m
        sdv[pl.ds(t * 16, 16)] = (invd + kvec) | m
        kpv[pl.ds(t * 16, 16)] = m + 1

    o0 = pltpu.async_copy(ssv, ss_hbm.at[pl.ds(ebase, EDGES_PW)], sem_o)
    o1 = pltpu.async_copy(sdv, sd_hbm.at[pl.ds(ebase, EDGES_PW)], sem_o)
    o2 = pltpu.async_copy(kpv, kp_hbm.at[pl.ds(ebase, EDGES_PW)], sem_o)
    o0.wait()
    o1.wait()
    o2.wait()


def _sc_gpool(h, ids_pad, vals_pad, ei_flat, ks16, interpret=False):
    mesh = plsc.VectorSubcoreMesh(
        core_axis_name="c", subcore_axis_name="s", num_cores=NC, num_subcores=NS)
    fn = functools.partial(
        pl.kernel,
        out_type=[
            jax.ShapeDtypeStruct((K, D), jnp.float32),
            jax.ShapeDtypeStruct((E,), jnp.int32),
            jax.ShapeDtypeStruct((E,), jnp.int32),
            jax.ShapeDtypeStruct((E,), jnp.int32),
        ],
        mesh=mesh,
        compiler_params=pltpu.CompilerParams(needs_layout_passes=False),
        scratch_types=[
            pltpu.VMEM((N,), jnp.int32),
            pltpu.VMEM((KPAD,), jnp.int32),
            pltpu.VMEM((2, 80), jnp.int32),
            pltpu.VMEM((ROWS_PW,), jnp.float32),
            pltpu.VMEM((ROWS_PW, D), jnp.float32),
            pltpu.VMEM((EDGES_PW,), jnp.int32),
            pltpu.VMEM((EDGES_PW,), jnp.int32),
            pltpu.VMEM((EDGES_PW,), jnp.int32),
            pltpu.VMEM((EDGES_PW,), jnp.int32),
            pltpu.VMEM((EDGES_PW,), jnp.int32),
            pltpu.VMEM((16,), jnp.int32),
            pltpu.SemaphoreType.DMA,
            pltpu.SemaphoreType.DMA,
            pltpu.SemaphoreType.DMA,
            pltpu.SemaphoreType.DMA,
            pltpu.SemaphoreType.DMA,
            pltpu.SemaphoreType.DMA,
        ],
        interpret=interpret,
    )(_sc_gpool_body)
    return fn(h, ids_pad, vals_pad, ei_flat, ks16)


def kernel(h, edge_index, top_k, W, b):
    scores = jax.nn.sigmoid(h @ W + b)[:, 0]
    k_shift = jnp.asarray(top_k, jnp.int32) - jnp.int32(K)

    spad = jnp.concatenate(
        [scores, jnp.full((NPAD - N,), -1.0, jnp.float32)]).reshape(128, 128)
    vals2d, ids2d = _topk_sort(spad)
    vals_lin = vals2d.reshape(-1)
    ids_lin = ids2d.reshape(-1)
    node_ids = ids_lin[:K]

    ks16 = jnp.full((16,), 1, jnp.int32) * k_shift
    new_h, ss, sd, kp = _sc_gpool(
        h, ids_lin[:KPAD], vals_lin[:KPAD], edge_index.reshape(-1), ks16)

    return (new_h, node_ids, ss, sd, kp.astype(jnp.bool_))
